# baseline (speedup 1.0000x reference)
import functools

import jax
import jax.numpy as jnp
import numpy as np
from jax.experimental import pallas as pl
from jax.experimental.pallas import tpu as pltpu

TB = 256
R_Y1, R_A1, R_Y2, R_A2 = 28 * 168, 14 * 96, 10 * 160, 5 * 80

_C1_BASES = [28 * min(max(2 * hp - 2, 0), 22) for hp in range(14)]


def _lenet_mxu_kernel(x_ref, w1a_ref, w1b_ref, w1c_ref, b1_ref, p1_ref,
                      w2_ref, b2_ref, p2_ref, w3_ref, b3_ref,
                      wf1_ref, bf1_ref, wf2_ref, bf2_ref, out_ref,
                      y1_ref, a1_ref, y2_ref, a2_ref):
    f32 = jnp.float32
    dot = functools.partial(jnp.dot, preferred_element_type=f32)

    for hp in range(14):
        m_ref = w1a_ref if hp == 0 else (w1c_ref if hp == 13 else w1b_ref)
        xs = x_ref[_C1_BASES[hp]:_C1_BASES[hp] + 168, :]
        y1_ref[hp * 336:(hp + 1) * 336, :] = jnp.tanh(
            dot(m_ref[...], xs) + b1_ref[...])

    for h2 in range(14):
        rs = (y1_ref[(2 * h2) * 168:(2 * h2 + 1) * 168, :]
              + y1_ref[(2 * h2 + 1) * 168:(2 * h2 + 2) * 168, :])
        a1_ref[h2 * 96:(h2 + 1) * 96, :] = dot(p1_ref[...], rs)

    for h in range(10):
        s = a1_ref[h * 96:h * 96 + 480, :]
        y2_ref[h * 160:(h + 1) * 160, :] = jnp.tanh(
            dot(w2_ref[...], s) + b2_ref[...])

    for h2 in range(5):
        rs = (y2_ref[(2 * h2) * 160:(2 * h2 + 1) * 160, :]
              + y2_ref[(2 * h2 + 1) * 160:(2 * h2 + 2) * 160, :])
        a2_ref[h2 * 80:(h2 + 1) * 80, :] = dot(p2_ref[...], rs)

    y3 = jnp.tanh(dot(w3_ref[...], a2_ref[...]) + b3_ref[...])
    hfc = jnp.tanh(dot(wf1_ref[...], y3) + bf1_ref[...])
    out_ref[...] = dot(wf2_ref[...], hfc) + bf2_ref[...]


def _np_conv1_placement(hp):
    base_row = _C1_BASES[hp] // 28
    dh, dw, hh, w = np.meshgrid(np.arange(5), np.arange(5), np.arange(2),
                                np.arange(28), indexing="ij")
    y = 2 * hp + hh + dh - 2
    x = w + dw - 2
    ok = (y >= 0) & (y <= 27) & (x >= 0) & (x <= 27)
    c = np.zeros((5, 5, 2, 28, 6, 28), np.float32)
    rb = np.clip(y - base_row, 0, 5)
    xc = np.clip(x, 0, 27)
    np.add.at(c, (dh, dw, hh, w, rb, xc), ok.astype(np.float32))
    return c


_C1_PLACE = {hp: _np_conv1_placement(hp) for hp in (0, 6, 13)}

_C2_PLACE = np.zeros((5, 10, 16), np.float32)
for _dw in range(5):
    for _w in range(10):
        _C2_PLACE[_dw, _w, _w + _dw] = 1.0

_P1M = np.zeros((96, 168), np.float32)
for _c in range(6):
    for _w2 in range(14):
        for _j in range(2):
            _P1M[_c * 16 + _w2, _c * 28 + 2 * _w2 + _j] = 0.25
_P2M = np.zeros((80, 160), np.float32)
for _o in range(16):
    for _w2 in range(5):
        for _j in range(2):
            _P2M[_o * 5 + _w2, _o * 10 + 2 * _w2 + _j] = 0.25


def _build_matrices(w1, b1, w2, b2, w3):
    w1r = w1.reshape(5, 5, 6)
    w1s = [jnp.einsum("dec,dehwrx->hcwrx", w1r, _C1_PLACE[hp]
                      ).reshape(336, 168) for hp in (0, 6, 13)]
    w1a, w1b, w1c = w1s
    b1v = jnp.tile(jnp.repeat(b1, 28), 2).reshape(336, 1)

    w2r = w2.reshape(5, 5, 6, 16)
    w2m = jnp.einsum("deco,ewx->owdcx", w2r, jnp.asarray(_C2_PLACE)
                     ).reshape(160, 480)
    b2v = jnp.repeat(b2, 10).reshape(160, 1)

    w3p = w3.reshape(120, 16, 5, 5).transpose(0, 2, 1, 3).reshape(120, 400)
    return w1a, w1b, w1c, b1v, jnp.asarray(_P1M), w2m, b2v, jnp.asarray(_P2M), w3p


@jax.jit
def _forward(w1, b1, w2, b2, w3, b3, wf1, bf1, wf2, bf2, x):
    B = x.shape[0]
    G = (B + TB - 1) // TB
    Bp = G * TB

    (w1a, w1b, w1c, b1v, p1m, w2m, b2v, p2m, w3p
     ) = _build_matrices(w1, b1, w2, b2, w3)

    xi = jnp.transpose(x, (1, 2, 3, 0)).reshape(28 * 28, B)
    if Bp != B:
        xi = jnp.pad(xi, ((0, 0), (0, Bp - B)))

    def fixed(a):
        zeros = (0,) * a.ndim
        return pl.BlockSpec(a.shape, lambda g, _z=zeros: _z)

    consts = (w1a, w1b, w1c, b1v, p1m, w2m, b2v, p2m, w3p,
              b3, wf1, bf1, wf2, bf2)

    out = pl.pallas_call(
        _lenet_mxu_kernel,
        out_shape=jax.ShapeDtypeStruct((10, Bp), jnp.float32),
        grid=(G,),
        in_specs=[pl.BlockSpec((28 * 28, TB), lambda g: (0, g))]
        + [fixed(a) for a in consts],
        out_specs=pl.BlockSpec((10, TB), lambda g: (0, g)),
        scratch_shapes=[
            pltpu.VMEM((R_Y1, TB), jnp.float32),
            pltpu.VMEM((R_A1, TB), jnp.float32),
            pltpu.VMEM((R_Y2, TB), jnp.float32),
            pltpu.VMEM((R_A2, TB), jnp.float32),
        ],
        compiler_params=pltpu.CompilerParams(
            dimension_semantics=("parallel",),
            vmem_limit_bytes=64 * 1024 * 1024),
        cost_estimate=pl.CostEstimate(
            flops=2 * Bp * (336 * 168 * 14 + 96 * 168 * 14 + 160 * 480 * 10
                            + 80 * 160 * 5 + 120 * 400 + 84 * 120 + 10 * 84),
            transcendentals=Bp * (R_Y1 + R_Y2 + 120 + 84),
            bytes_accessed=int(Bp * 784 * 4 + Bp * 10 * 4 + 600 * 1024)),
    )(xi, *consts)

    return out.T[:B]


def kernel(w1, b1, w2, b2, w3, b3, wf1, bf1, wf2, bf2, d1, d2, x):
    del d1, d2
    return _forward(w1, b1, w2, b2, w3, b3, wf1, bf1, wf2, bf2, x)


# speedup vs baseline: 1.0308x; 1.0308x over previous
import functools

import jax
import jax.numpy as jnp
import numpy as np
from jax.experimental import pallas as pl
from jax.experimental.pallas import tpu as pltpu

TB = 256
R_Y1, R_A1, R_Y2, R_A2 = 28 * 168, 14 * 96, 10 * 160, 5 * 80

_C1_BASES = [28 * min(max(2 * hp - 2, 0), 22) for hp in range(14)]


def _lenet_mxu_kernel(x_ref, w1a_ref, w1b_ref, w1c_ref, b1_ref, p1_ref,
                      w2_ref, b2_ref, p2_ref, w3_ref, b3_ref,
                      wf1_ref, bf1_ref, wf2_ref, bf2_ref, out_ref,
                      y1_ref, a1_ref, y2_ref, a2_ref):
    f32 = jnp.float32
    dot = functools.partial(jnp.dot, preferred_element_type=f32)

    for hp in range(14):
        m_ref = w1a_ref if hp == 0 else (w1c_ref if hp == 13 else w1b_ref)
        xs = x_ref[_C1_BASES[hp]:_C1_BASES[hp] + 168, :]
        y1_ref[hp * 336:(hp + 1) * 336, :] = jnp.tanh(
            dot(m_ref[...], xs) + b1_ref[...])

    for h2 in range(14):
        rs = (y1_ref[(2 * h2) * 168:(2 * h2 + 1) * 168, :]
              + y1_ref[(2 * h2 + 1) * 168:(2 * h2 + 2) * 168, :])
        a1_ref[h2 * 96:(h2 + 1) * 96, :] = dot(p1_ref[...], rs)

    for h in range(10):
        s = a1_ref[h * 96:h * 96 + 480, :]
        y2_ref[h * 160:(h + 1) * 160, :] = jnp.tanh(
            dot(w2_ref[...], s) + b2_ref[...])

    for h2 in range(5):
        rs = (y2_ref[(2 * h2) * 160:(2 * h2 + 1) * 160, :]
              + y2_ref[(2 * h2 + 1) * 160:(2 * h2 + 2) * 160, :])
        a2_ref[h2 * 80:(h2 + 1) * 80, :] = dot(p2_ref[...], rs)

    y3 = jnp.tanh(dot(w3_ref[...], a2_ref[...]) + b3_ref[...])
    hfc = jnp.tanh(dot(wf1_ref[...], y3) + bf1_ref[...])
    out_ref[...] = dot(wf2_ref[...], hfc) + bf2_ref[...]


def _np_conv1_placement(hp):
    base_row = _C1_BASES[hp] // 28
    dh, dw, hh, w = np.meshgrid(np.arange(5), np.arange(5), np.arange(2),
                                np.arange(28), indexing="ij")
    y = 2 * hp + hh + dh - 2
    x = w + dw - 2
    ok = (y >= 0) & (y <= 27) & (x >= 0) & (x <= 27)
    c = np.zeros((5, 5, 2, 28, 6, 28), np.float32)
    rb = np.clip(y - base_row, 0, 5)
    xc = np.clip(x, 0, 27)
    np.add.at(c, (dh, dw, hh, w, rb, xc), ok.astype(np.float32))
    return c


_C1_PLACE = {hp: _np_conv1_placement(hp) for hp in (0, 6, 13)}

_C2_PLACE = np.zeros((5, 10, 16), np.float32)
for _dw in range(5):
    for _w in range(10):
        _C2_PLACE[_dw, _w, _w + _dw] = 1.0

_P1M = np.zeros((96, 168), np.float32)
for _c in range(6):
    for _w2 in range(14):
        for _j in range(2):
            _P1M[_c * 16 + _w2, _c * 28 + 2 * _w2 + _j] = 0.25
_P2M = np.zeros((80, 160), np.float32)
for _o in range(16):
    for _w2 in range(5):
        for _j in range(2):
            _P2M[_o * 5 + _w2, _o * 10 + 2 * _w2 + _j] = 0.25


def _build_matrices(w1, b1, w2, b2, w3):
    w1r = w1.reshape(5, 5, 6)
    w1s = [jnp.einsum("dec,dehwrx->hcwrx", w1r, _C1_PLACE[hp]
                      ).reshape(336, 168).astype(jnp.bfloat16)
           for hp in (0, 6, 13)]
    w1a, w1b, w1c = w1s
    b1v = jnp.tile(jnp.repeat(b1, 28), 2).reshape(336, 1)

    w2r = w2.reshape(5, 5, 6, 16)
    w2m = jnp.einsum("deco,ewx->owdcx", w2r, jnp.asarray(_C2_PLACE)
                     ).reshape(160, 480)
    b2v = jnp.repeat(b2, 10).reshape(160, 1)

    w3p = w3.reshape(120, 16, 5, 5).transpose(0, 2, 1, 3).reshape(120, 400)
    return w1a, w1b, w1c, b1v, jnp.asarray(_P1M), w2m, b2v, jnp.asarray(_P2M), w3p


@jax.jit
def _forward(w1, b1, w2, b2, w3, b3, wf1, bf1, wf2, bf2, x):
    B = x.shape[0]
    G = (B + TB - 1) // TB
    Bp = G * TB

    (w1a, w1b, w1c, b1v, p1m, w2m, b2v, p2m, w3p
     ) = _build_matrices(w1, b1, w2, b2, w3)

    xi = x.astype(jnp.bfloat16).reshape(B, 28 * 28)
    if Bp != B:
        xi = jnp.pad(xi, ((0, Bp - B), (0, 0)))
    xi = xi.T

    def fixed(a):
        zeros = (0,) * a.ndim
        return pl.BlockSpec(a.shape, lambda g, _z=zeros: _z)

    consts = (w1a, w1b, w1c, b1v, p1m, w2m, b2v, p2m, w3p,
              b3, wf1, bf1, wf2, bf2)

    out = pl.pallas_call(
        _lenet_mxu_kernel,
        out_shape=jax.ShapeDtypeStruct((10, Bp), jnp.float32),
        grid=(G,),
        in_specs=[pl.BlockSpec((28 * 28, TB), lambda g: (0, g))]
        + [fixed(a) for a in consts],
        out_specs=pl.BlockSpec((10, TB), lambda g: (0, g)),
        scratch_shapes=[
            pltpu.VMEM((R_Y1, TB), jnp.float32),
            pltpu.VMEM((R_A1, TB), jnp.float32),
            pltpu.VMEM((R_Y2, TB), jnp.float32),
            pltpu.VMEM((R_A2, TB), jnp.float32),
        ],
        compiler_params=pltpu.CompilerParams(
            dimension_semantics=("parallel",),
            vmem_limit_bytes=64 * 1024 * 1024),
        cost_estimate=pl.CostEstimate(
            flops=2 * Bp * (336 * 168 * 14 + 96 * 168 * 14 + 160 * 480 * 10
                            + 80 * 160 * 5 + 120 * 400 + 84 * 120 + 10 * 84),
            transcendentals=Bp * (R_Y1 + R_Y2 + 120 + 84),
            bytes_accessed=int(Bp * 784 * 4 + Bp * 10 * 4 + 600 * 1024)),
    )(xi, *consts)

    return out.T[:B]


def kernel(w1, b1, w2, b2, w3, b3, wf1, bf1, wf2, bf2, d1, d2, x):
    del d1, d2
    return _forward(w1, b1, w2, b2, w3, b3, wf1, bf1, wf2, bf2, x)


# speedup vs baseline: 1.1454x; 1.1112x over previous
import functools

import jax
import jax.numpy as jnp
import numpy as np
from jax.experimental import pallas as pl
from jax.experimental.pallas import tpu as pltpu

TB = 512
R_Y1, R_A1, R_Y2, R_A2 = 28 * 168, 14 * 96, 10 * 160, 5 * 80

_C1_BASES = [28 * min(max(2 * hp - 2, 0), 22) for hp in range(14)]


def _lenet_mxu_kernel(x_ref, w1a_ref, w1b_ref, w1c_ref, b1_ref, p1_ref,
                      w2_ref, b2_ref, p2_ref, w3_ref, b3_ref,
                      wf1_ref, bf1_ref, wf2_ref, bf2_ref, out_ref,
                      y1_ref, a1_ref, y2_ref, a2_ref):
    f32 = jnp.float32
    dot = functools.partial(jnp.dot, preferred_element_type=f32)

    for hp in range(14):
        m_ref = w1a_ref if hp == 0 else (w1c_ref if hp == 13 else w1b_ref)
        xs = x_ref[_C1_BASES[hp]:_C1_BASES[hp] + 168, :]
        y1_ref[hp * 336:(hp + 1) * 336, :] = jnp.tanh(
            dot(m_ref[...], xs) + b1_ref[...])

    for h2 in range(14):
        rs = (y1_ref[(2 * h2) * 168:(2 * h2 + 1) * 168, :]
              + y1_ref[(2 * h2 + 1) * 168:(2 * h2 + 2) * 168, :])
        a1_ref[h2 * 96:(h2 + 1) * 96, :] = dot(p1_ref[...], rs)

    for h in range(10):
        s = a1_ref[h * 96:h * 96 + 480, :]
        y2_ref[h * 160:(h + 1) * 160, :] = jnp.tanh(
            dot(w2_ref[...], s) + b2_ref[...])

    for h2 in range(5):
        rs = (y2_ref[(2 * h2) * 160:(2 * h2 + 1) * 160, :]
              + y2_ref[(2 * h2 + 1) * 160:(2 * h2 + 2) * 160, :])
        a2_ref[h2 * 80:(h2 + 1) * 80, :] = dot(p2_ref[...], rs)

    y3 = jnp.tanh(dot(w3_ref[...], a2_ref[...]) + b3_ref[...])
    hfc = jnp.tanh(dot(wf1_ref[...], y3) + bf1_ref[...])
    out_ref[...] = dot(wf2_ref[...], hfc) + bf2_ref[...]


def _np_conv1_placement(hp):
    base_row = _C1_BASES[hp] // 28
    dh, dw, hh, w = np.meshgrid(np.arange(5), np.arange(5), np.arange(2),
                                np.arange(28), indexing="ij")
    y = 2 * hp + hh + dh - 2
    x = w + dw - 2
    ok = (y >= 0) & (y <= 27) & (x >= 0) & (x <= 27)
    c = np.zeros((5, 5, 2, 28, 6, 28), np.float32)
    rb = np.clip(y - base_row, 0, 5)
    xc = np.clip(x, 0, 27)
    np.add.at(c, (dh, dw, hh, w, rb, xc), ok.astype(np.float32))
    return c


_C1_PLACE = {hp: _np_conv1_placement(hp) for hp in (0, 6, 13)}

_C2_PLACE = np.zeros((5, 10, 16), np.float32)
for _dw in range(5):
    for _w in range(10):
        _C2_PLACE[_dw, _w, _w + _dw] = 1.0

_P1M = np.zeros((96, 168), np.float32)
for _c in range(6):
    for _w2 in range(14):
        for _j in range(2):
            _P1M[_c * 16 + _w2, _c * 28 + 2 * _w2 + _j] = 0.25
_P2M = np.zeros((80, 160), np.float32)
for _o in range(16):
    for _w2 in range(5):
        for _j in range(2):
            _P2M[_o * 5 + _w2, _o * 10 + 2 * _w2 + _j] = 0.25


def _build_matrices(w1, b1, w2, b2, w3):
    w1r = w1.reshape(5, 5, 6)
    w1s = [jnp.einsum("dec,dehwrx->hcwrx", w1r, _C1_PLACE[hp]
                      ).reshape(336, 168).astype(jnp.bfloat16)
           for hp in (0, 6, 13)]
    w1a, w1b, w1c = w1s
    b1v = jnp.tile(jnp.repeat(b1, 28), 2).reshape(336, 1)

    w2r = w2.reshape(5, 5, 6, 16)
    w2m = jnp.einsum("deco,ewx->owdcx", w2r, jnp.asarray(_C2_PLACE)
                     ).reshape(160, 480)
    b2v = jnp.repeat(b2, 10).reshape(160, 1)

    w3p = w3.reshape(120, 16, 5, 5).transpose(0, 2, 1, 3).reshape(120, 400)
    return w1a, w1b, w1c, b1v, jnp.asarray(_P1M), w2m, b2v, jnp.asarray(_P2M), w3p


@jax.jit
def _forward(w1, b1, w2, b2, w3, b3, wf1, bf1, wf2, bf2, x):
    B = x.shape[0]
    G = (B + TB - 1) // TB
    Bp = G * TB

    (w1a, w1b, w1c, b1v, p1m, w2m, b2v, p2m, w3p
     ) = _build_matrices(w1, b1, w2, b2, w3)

    xi = x.astype(jnp.bfloat16).reshape(B, 28 * 28)
    if Bp != B:
        xi = jnp.pad(xi, ((0, Bp - B), (0, 0)))
    xi = xi.T

    def fixed(a):
        zeros = (0,) * a.ndim
        return pl.BlockSpec(a.shape, lambda g, _z=zeros: _z)

    consts = (w1a, w1b, w1c, b1v, p1m, w2m, b2v, p2m, w3p,
              b3, wf1, bf1, wf2, bf2)

    out = pl.pallas_call(
        _lenet_mxu_kernel,
        out_shape=jax.ShapeDtypeStruct((10, Bp), jnp.float32),
        grid=(G,),
        in_specs=[pl.BlockSpec((28 * 28, TB), lambda g: (0, g))]
        + [fixed(a) for a in consts],
        out_specs=pl.BlockSpec((10, TB), lambda g: (0, g)),
        scratch_shapes=[
            pltpu.VMEM((R_Y1, TB), jnp.float32),
            pltpu.VMEM((R_A1, TB), jnp.float32),
            pltpu.VMEM((R_Y2, TB), jnp.float32),
            pltpu.VMEM((R_A2, TB), jnp.float32),
        ],
        compiler_params=pltpu.CompilerParams(
            dimension_semantics=("parallel",),
            vmem_limit_bytes=64 * 1024 * 1024),
        cost_estimate=pl.CostEstimate(
            flops=2 * Bp * (336 * 168 * 14 + 96 * 168 * 14 + 160 * 480 * 10
                            + 80 * 160 * 5 + 120 * 400 + 84 * 120 + 10 * 84),
            transcendentals=Bp * (R_Y1 + R_Y2 + 120 + 84),
            bytes_accessed=int(Bp * 784 * 4 + Bp * 10 * 4 + 600 * 1024)),
    )(xi, *consts)

    return out.T[:B]


def kernel(w1, b1, w2, b2, w3, b3, wf1, bf1, wf2, bf2, d1, d2, x):
    del d1, d2
    return _forward(w1, b1, w2, b2, w3, b3, wf1, bf1, wf2, bf2, x)


# speedup vs baseline: 1.2004x; 1.0479x over previous
import functools

import jax
import jax.numpy as jnp
import numpy as np
from jax.experimental import pallas as pl
from jax.experimental.pallas import tpu as pltpu

TB = 1024
R_Y1, R_A1, R_Y2, R_A2 = 28 * 168, 14 * 96, 10 * 160, 5 * 80

_C1_BASES = [28 * min(max(2 * hp - 2, 0), 22) for hp in range(14)]


def _lenet_mxu_kernel(x_ref, w1a_ref, w1b_ref, w1c_ref, b1_ref, p1_ref,
                      w2_ref, b2_ref, p2_ref, w3_ref, b3_ref,
                      wf1_ref, bf1_ref, wf2_ref, bf2_ref, out_ref,
                      y1_ref, a1_ref, y2_ref, a2_ref):
    f32 = jnp.float32
    dot = functools.partial(jnp.dot, preferred_element_type=f32)

    for hp in range(14):
        m_ref = w1a_ref if hp == 0 else (w1c_ref if hp == 13 else w1b_ref)
        xs = x_ref[_C1_BASES[hp]:_C1_BASES[hp] + 168, :]
        y1_ref[hp * 336:(hp + 1) * 336, :] = jnp.tanh(
            dot(m_ref[...], xs) + b1_ref[...])

    for h2 in range(14):
        rs = (y1_ref[(2 * h2) * 168:(2 * h2 + 1) * 168, :]
              + y1_ref[(2 * h2 + 1) * 168:(2 * h2 + 2) * 168, :])
        a1_ref[h2 * 96:(h2 + 1) * 96, :] = dot(p1_ref[...], rs)

    for h in range(10):
        s = a1_ref[h * 96:h * 96 + 480, :]
        y2_ref[h * 160:(h + 1) * 160, :] = jnp.tanh(
            dot(w2_ref[...], s) + b2_ref[...])

    for h2 in range(5):
        rs = (y2_ref[(2 * h2) * 160:(2 * h2 + 1) * 160, :]
              + y2_ref[(2 * h2 + 1) * 160:(2 * h2 + 2) * 160, :])
        a2_ref[h2 * 80:(h2 + 1) * 80, :] = dot(p2_ref[...], rs)

    y3 = jnp.tanh(dot(w3_ref[...], a2_ref[...]) + b3_ref[...])
    hfc = jnp.tanh(dot(wf1_ref[...], y3) + bf1_ref[...])
    out_ref[...] = dot(wf2_ref[...], hfc) + bf2_ref[...]


def _np_conv1_placement(hp):
    base_row = _C1_BASES[hp] // 28
    dh, dw, hh, w = np.meshgrid(np.arange(5), np.arange(5), np.arange(2),
                                np.arange(28), indexing="ij")
    y = 2 * hp + hh + dh - 2
    x = w + dw - 2
    ok = (y >= 0) & (y <= 27) & (x >= 0) & (x <= 27)
    c = np.zeros((5, 5, 2, 28, 6, 28), np.float32)
    rb = np.clip(y - base_row, 0, 5)
    xc = np.clip(x, 0, 27)
    np.add.at(c, (dh, dw, hh, w, rb, xc), ok.astype(np.float32))
    return c


_C1_PLACE = {hp: _np_conv1_placement(hp) for hp in (0, 6, 13)}

_C2_PLACE = np.zeros((5, 10, 16), np.float32)
for _dw in range(5):
    for _w in range(10):
        _C2_PLACE[_dw, _w, _w + _dw] = 1.0

_P1M = np.zeros((96, 168), np.float32)
for _c in range(6):
    for _w2 in range(14):
        for _j in range(2):
            _P1M[_c * 16 + _w2, _c * 28 + 2 * _w2 + _j] = 0.25
_P2M = np.zeros((80, 160), np.float32)
for _o in range(16):
    for _w2 in range(5):
        for _j in range(2):
            _P2M[_o * 5 + _w2, _o * 10 + 2 * _w2 + _j] = 0.25


def _build_matrices(w1, b1, w2, b2, w3):
    w1r = w1.reshape(5, 5, 6)
    w1s = [jnp.einsum("dec,dehwrx->hcwrx", w1r, _C1_PLACE[hp]
                      ).reshape(336, 168).astype(jnp.bfloat16)
           for hp in (0, 6, 13)]
    w1a, w1b, w1c = w1s
    b1v = jnp.tile(jnp.repeat(b1, 28), 2).reshape(336, 1)

    w2r = w2.reshape(5, 5, 6, 16)
    w2m = jnp.einsum("deco,ewx->owdcx", w2r, jnp.asarray(_C2_PLACE)
                     ).reshape(160, 480)
    b2v = jnp.repeat(b2, 10).reshape(160, 1)

    w3p = w3.reshape(120, 16, 5, 5).transpose(0, 2, 1, 3).reshape(120, 400)
    return w1a, w1b, w1c, b1v, jnp.asarray(_P1M), w2m, b2v, jnp.asarray(_P2M), w3p


@jax.jit
def _forward(w1, b1, w2, b2, w3, b3, wf1, bf1, wf2, bf2, x):
    B = x.shape[0]
    G = (B + TB - 1) // TB
    Bp = G * TB

    (w1a, w1b, w1c, b1v, p1m, w2m, b2v, p2m, w3p
     ) = _build_matrices(w1, b1, w2, b2, w3)

    xi = x.astype(jnp.bfloat16).reshape(B, 28 * 28)
    if Bp != B:
        xi = jnp.pad(xi, ((0, Bp - B), (0, 0)))
    xi = xi.T

    def fixed(a):
        zeros = (0,) * a.ndim
        return pl.BlockSpec(a.shape, lambda g, _z=zeros: _z)

    consts = (w1a, w1b, w1c, b1v, p1m, w2m, b2v, p2m, w3p,
              b3, wf1, bf1, wf2, bf2)

    out = pl.pallas_call(
        _lenet_mxu_kernel,
        out_shape=jax.ShapeDtypeStruct((10, Bp), jnp.float32),
        grid=(G,),
        in_specs=[pl.BlockSpec((28 * 28, TB), lambda g: (0, g))]
        + [fixed(a) for a in consts],
        out_specs=pl.BlockSpec((10, TB), lambda g: (0, g)),
        scratch_shapes=[
            pltpu.VMEM((R_Y1, TB), jnp.float32),
            pltpu.VMEM((R_A1, TB), jnp.float32),
            pltpu.VMEM((R_Y2, TB), jnp.float32),
            pltpu.VMEM((R_A2, TB), jnp.float32),
        ],
        compiler_params=pltpu.CompilerParams(
            dimension_semantics=("parallel",),
            vmem_limit_bytes=64 * 1024 * 1024),
        cost_estimate=pl.CostEstimate(
            flops=2 * Bp * (336 * 168 * 14 + 96 * 168 * 14 + 160 * 480 * 10
                            + 80 * 160 * 5 + 120 * 400 + 84 * 120 + 10 * 84),
            transcendentals=Bp * (R_Y1 + R_Y2 + 120 + 84),
            bytes_accessed=int(Bp * 784 * 4 + Bp * 10 * 4 + 600 * 1024)),
    )(xi, *consts)

    return out.T[:B]


def kernel(w1, b1, w2, b2, w3, b3, wf1, bf1, wf2, bf2, d1, d2, x):
    del d1, d2
    return _forward(w1, b1, w2, b2, w3, b3, wf1, bf1, wf2, bf2, x)


# speedup vs baseline: 1.9513x; 1.6256x over previous
import functools

import jax
import jax.numpy as jnp
import numpy as np
from jax.experimental import pallas as pl
from jax.experimental.pallas import tpu as pltpu

TB = 1024
R_Y1, R_A1, R_Y2, R_A2 = 28 * 168, 14 * 96, 10 * 160, 5 * 80

_C1_BASES = [28 * min(max(2 * hp - 2, 0), 22) for hp in range(14)]


def _lenet_mxu_kernel(x_ref, w1a_ref, w1b_ref, w1c_ref, b1_ref, p1_ref,
                      w2_ref, b2_ref, p2_ref, w3_ref, b3_ref,
                      wf1_ref, bf1_ref, wf2_ref, bf2_ref, out_ref,
                      xg_ref, y1_ref, a1_ref, y2_ref, a2_ref):
    f32 = jnp.float32
    dot = functools.partial(jnp.dot, preferred_element_type=f32)

    for y in range(28):
        xg_ref[y * 28:(y + 1) * 28, :] = x_ref[0, y]

    for hp in range(14):
        m_ref = w1a_ref if hp == 0 else (w1c_ref if hp == 13 else w1b_ref)
        xs = xg_ref[_C1_BASES[hp]:_C1_BASES[hp] + 168, :]
        y1_ref[hp * 336:(hp + 1) * 336, :] = jnp.tanh(
            dot(m_ref[...], xs) + b1_ref[...])

    for h2 in range(14):
        rs = (y1_ref[(2 * h2) * 168:(2 * h2 + 1) * 168, :]
              + y1_ref[(2 * h2 + 1) * 168:(2 * h2 + 2) * 168, :])
        a1_ref[h2 * 96:(h2 + 1) * 96, :] = dot(p1_ref[...], rs)

    for h in range(10):
        s = a1_ref[h * 96:h * 96 + 480, :]
        y2_ref[h * 160:(h + 1) * 160, :] = jnp.tanh(
            dot(w2_ref[...], s) + b2_ref[...])

    for h2 in range(5):
        rs = (y2_ref[(2 * h2) * 160:(2 * h2 + 1) * 160, :]
              + y2_ref[(2 * h2 + 1) * 160:(2 * h2 + 2) * 160, :])
        a2_ref[h2 * 80:(h2 + 1) * 80, :] = dot(p2_ref[...], rs)

    y3 = jnp.tanh(dot(w3_ref[...], a2_ref[...]) + b3_ref[...])
    hfc = jnp.tanh(dot(wf1_ref[...], y3) + bf1_ref[...])
    out_ref[...] = dot(wf2_ref[...], hfc) + bf2_ref[...]


def _np_conv1_placement(hp):
    base_row = _C1_BASES[hp] // 28
    dh, dw, hh, w = np.meshgrid(np.arange(5), np.arange(5), np.arange(2),
                                np.arange(28), indexing="ij")
    y = 2 * hp + hh + dh - 2
    x = w + dw - 2
    ok = (y >= 0) & (y <= 27) & (x >= 0) & (x <= 27)
    c = np.zeros((5, 5, 2, 28, 6, 28), np.float32)
    rb = np.clip(y - base_row, 0, 5)
    xc = np.clip(x, 0, 27)
    np.add.at(c, (dh, dw, hh, w, rb, xc), ok.astype(np.float32))
    return c


_C1_PLACE = {hp: _np_conv1_placement(hp) for hp in (0, 6, 13)}

_C2_PLACE = np.zeros((5, 10, 16), np.float32)
for _dw in range(5):
    for _w in range(10):
        _C2_PLACE[_dw, _w, _w + _dw] = 1.0

_P1M = np.zeros((96, 168), np.float32)
for _c in range(6):
    for _w2 in range(14):
        for _j in range(2):
            _P1M[_c * 16 + _w2, _c * 28 + 2 * _w2 + _j] = 0.25
_P2M = np.zeros((80, 160), np.float32)
for _o in range(16):
    for _w2 in range(5):
        for _j in range(2):
            _P2M[_o * 5 + _w2, _o * 10 + 2 * _w2 + _j] = 0.25


def _build_matrices(w1, b1, w2, b2, w3):
    w1r = w1.reshape(5, 5, 6)
    w1s = [jnp.einsum("dec,dehwrx->hcwrx", w1r, _C1_PLACE[hp]
                      ).reshape(336, 168).astype(jnp.bfloat16)
           for hp in (0, 6, 13)]
    w1a, w1b, w1c = w1s
    b1v = jnp.tile(jnp.repeat(b1, 28), 2).reshape(336, 1)

    w2r = w2.reshape(5, 5, 6, 16)
    w2m = jnp.einsum("deco,ewx->owdcx", w2r, jnp.asarray(_C2_PLACE)
                     ).reshape(160, 480)
    b2v = jnp.repeat(b2, 10).reshape(160, 1)

    w3p = w3.reshape(120, 16, 5, 5).transpose(0, 2, 1, 3).reshape(120, 400)
    return w1a, w1b, w1c, b1v, jnp.asarray(_P1M), w2m, b2v, jnp.asarray(_P2M), w3p


@jax.jit
def _forward(w1, b1, w2, b2, w3, b3, wf1, bf1, wf2, bf2, x):
    B = x.shape[0]
    G = (B + TB - 1) // TB
    Bp = G * TB

    (w1a, w1b, w1c, b1v, p1m, w2m, b2v, p2m, w3p
     ) = _build_matrices(w1, b1, w2, b2, w3)

    xi = jnp.transpose(x.astype(jnp.bfloat16), (1, 2, 3, 0))
    if Bp != B:
        xi = jnp.pad(xi, ((0, 0), (0, 0), (0, 0), (0, Bp - B)))

    def fixed(a):
        zeros = (0,) * a.ndim
        return pl.BlockSpec(a.shape, lambda g, _z=zeros: _z)

    consts = (w1a, w1b, w1c, b1v, p1m, w2m, b2v, p2m, w3p,
              b3, wf1, bf1, wf2, bf2)

    out = pl.pallas_call(
        _lenet_mxu_kernel,
        out_shape=jax.ShapeDtypeStruct((10, Bp), jnp.float32),
        grid=(G,),
        in_specs=[pl.BlockSpec((1, 28, 28, TB), lambda g: (0, 0, 0, g))]
        + [fixed(a) for a in consts],
        out_specs=pl.BlockSpec((10, TB), lambda g: (0, g)),
        scratch_shapes=[
            pltpu.VMEM((28 * 28, TB), jnp.bfloat16),
            pltpu.VMEM((R_Y1, TB), jnp.float32),
            pltpu.VMEM((R_A1, TB), jnp.float32),
            pltpu.VMEM((R_Y2, TB), jnp.float32),
            pltpu.VMEM((R_A2, TB), jnp.float32),
        ],
        compiler_params=pltpu.CompilerParams(
            dimension_semantics=("parallel",),
            vmem_limit_bytes=64 * 1024 * 1024),
        cost_estimate=pl.CostEstimate(
            flops=2 * Bp * (336 * 168 * 14 + 96 * 168 * 14 + 160 * 480 * 10
                            + 80 * 160 * 5 + 120 * 400 + 84 * 120 + 10 * 84),
            transcendentals=Bp * (R_Y1 + R_Y2 + 120 + 84),
            bytes_accessed=int(Bp * 784 * 4 + Bp * 10 * 4 + 600 * 1024)),
    )(xi, *consts)

    return out.T[:B]


def kernel(w1, b1, w2, b2, w3, b3, wf1, bf1, wf2, bf2, d1, d2, x):
    del d1, d2
    return _forward(w1, b1, w2, b2, w3, b3, wf1, bf1, wf2, bf2, x)


# speedup vs baseline: 1.9943x; 1.0220x over previous
import functools

import jax
import jax.numpy as jnp
import numpy as np
from jax.experimental import pallas as pl
from jax.experimental.pallas import tpu as pltpu

TB = 1024
R_Y1, R_A1, R_Y2, R_A2 = 28 * 168, 14 * 96, 10 * 160, 5 * 80

_C1_BASES = [28 * min(max(2 * hp - 2, 0), 22) for hp in range(14)]


def _lenet_mxu_kernel(x_ref, w1s_ref, b1_ref, p1_ref,
                      w2_ref, b2_ref, p2_ref, w3_ref, b3_ref,
                      wf1_ref, bf1_ref, wf2_ref, bf2_ref, out_ref,
                      xg_ref, y1_ref, a1_ref, y2_ref, a2_ref):
    f32 = jnp.float32
    bf16 = jnp.bfloat16
    dot = functools.partial(jnp.dot, preferred_element_type=f32)

    for y in range(28):
        xg_ref[y * 28:(y + 1) * 28, :] = x_ref[0, y]

    for hp in range(14):
        m_ref = w1s_ref[0 if hp == 0 else (2 if hp == 13 else 1)]
        xs = xg_ref[_C1_BASES[hp]:_C1_BASES[hp] + 168, :]
        y1_ref[hp * 336:(hp + 1) * 336, :] = jnp.tanh(
            dot(m_ref, xs) + b1_ref[...])

    for h2 in range(14):
        rs = (y1_ref[(2 * h2) * 168:(2 * h2 + 1) * 168, :]
              + y1_ref[(2 * h2 + 1) * 168:(2 * h2 + 2) * 168, :])
        a1_ref[h2 * 96:(h2 + 1) * 96, :] = dot(p1_ref[...], rs).astype(bf16)

    for h in range(10):
        s = a1_ref[h * 96:h * 96 + 480, :]
        y2_ref[h * 160:(h + 1) * 160, :] = jnp.tanh(
            dot(w2_ref[...], s) + b2_ref[...])

    for h2 in range(5):
        rs = (y2_ref[(2 * h2) * 160:(2 * h2 + 1) * 160, :]
              + y2_ref[(2 * h2 + 1) * 160:(2 * h2 + 2) * 160, :])
        a2_ref[h2 * 80:(h2 + 1) * 80, :] = dot(p2_ref[...], rs)

    y3 = jnp.tanh(dot(w3_ref[...], a2_ref[...]) + b3_ref[...])
    hfc = jnp.tanh(dot(wf1_ref[...], y3) + bf1_ref[...])
    out_ref[...] = dot(wf2_ref[...], hfc) + bf2_ref[...]


def _np_conv1_placement(hp):
    base_row = _C1_BASES[hp] // 28
    dh, dw, hh, w = np.meshgrid(np.arange(5), np.arange(5), np.arange(2),
                                np.arange(28), indexing="ij")
    y = 2 * hp + hh + dh - 2
    x = w + dw - 2
    ok = (y >= 0) & (y <= 27) & (x >= 0) & (x <= 27)
    c = np.zeros((5, 5, 2, 28, 6, 28), np.float32)
    rb = np.clip(y - base_row, 0, 5)
    xc = np.clip(x, 0, 27)
    np.add.at(c, (dh, dw, hh, w, rb, xc), ok.astype(np.float32))
    return c


_C1_PLACE = np.stack([_np_conv1_placement(hp) for hp in (0, 6, 13)])

_C2_PLACE = np.zeros((5, 10, 16), np.float32)
for _dw in range(5):
    for _w in range(10):
        _C2_PLACE[_dw, _w, _w + _dw] = 1.0

_P1M = np.zeros((96, 168), np.float32)
for _c in range(6):
    for _w2 in range(14):
        for _j in range(2):
            _P1M[_c * 16 + _w2, _c * 28 + 2 * _w2 + _j] = 0.25
_P2M = np.zeros((80, 160), np.float32)
for _o in range(16):
    for _w2 in range(5):
        for _j in range(2):
            _P2M[_o * 5 + _w2, _o * 10 + 2 * _w2 + _j] = 0.25


def _build_matrices(w1, b1, w2, b2, w3):
    w1r = w1.reshape(5, 5, 6)
    w1s = jnp.einsum("dec,pdehwrx->phcwrx", w1r, jnp.asarray(_C1_PLACE)
                     ).reshape(3, 336, 168).astype(jnp.bfloat16)
    b1v = jnp.tile(jnp.repeat(b1, 28), 2).reshape(336, 1)

    w2r = w2.reshape(5, 5, 6, 16)
    w2m = jnp.einsum("deco,ewx->owdcx", w2r, jnp.asarray(_C2_PLACE)
                     ).reshape(160, 480).astype(jnp.bfloat16)
    b2v = jnp.repeat(b2, 10).reshape(160, 1)

    w3p = w3.reshape(120, 16, 5, 5).transpose(0, 2, 1, 3).reshape(120, 400)
    return w1s, b1v, jnp.asarray(_P1M), w2m, b2v, jnp.asarray(_P2M), w3p


@jax.jit
def _forward(w1, b1, w2, b2, w3, b3, wf1, bf1, wf2, bf2, x):
    B = x.shape[0]
    G = (B + TB - 1) // TB
    Bp = G * TB

    (w1s, b1v, p1m, w2m, b2v, p2m, w3p
     ) = _build_matrices(w1, b1, w2, b2, w3)

    xi = jnp.transpose(x.astype(jnp.bfloat16), (1, 2, 3, 0))
    if Bp != B:
        xi = jnp.pad(xi, ((0, 0), (0, 0), (0, 0), (0, Bp - B)))

    def fixed(a):
        zeros = (0,) * a.ndim
        return pl.BlockSpec(a.shape, lambda g, _z=zeros: _z)

    consts = (w1s, b1v, p1m, w2m, b2v, p2m, w3p,
              b3, wf1, bf1, wf2, bf2)

    out = pl.pallas_call(
        _lenet_mxu_kernel,
        out_shape=jax.ShapeDtypeStruct((10, Bp), jnp.float32),
        grid=(G,),
        in_specs=[pl.BlockSpec((1, 28, 28, TB), lambda g: (0, 0, 0, g))]
        + [fixed(a) for a in consts],
        out_specs=pl.BlockSpec((10, TB), lambda g: (0, g)),
        scratch_shapes=[
            pltpu.VMEM((28 * 28, TB), jnp.bfloat16),
            pltpu.VMEM((R_Y1, TB), jnp.float32),
            pltpu.VMEM((R_A1, TB), jnp.bfloat16),
            pltpu.VMEM((R_Y2, TB), jnp.float32),
            pltpu.VMEM((R_A2, TB), jnp.float32),
        ],
        compiler_params=pltpu.CompilerParams(
            dimension_semantics=("parallel",),
            vmem_limit_bytes=64 * 1024 * 1024),
        cost_estimate=pl.CostEstimate(
            flops=2 * Bp * (336 * 168 * 14 + 96 * 168 * 14 + 160 * 480 * 10
                            + 80 * 160 * 5 + 120 * 400 + 84 * 120 + 10 * 84),
            transcendentals=Bp * (R_Y1 + R_Y2 + 120 + 84),
            bytes_accessed=int(Bp * 784 * 4 + Bp * 10 * 4 + 600 * 1024)),
    )(xi, *consts)

    return out.T[:B]


def kernel(w1, b1, w2, b2, w3, b3, wf1, bf1, wf2, bf2, d1, d2, x):
    del d1, d2
    return _forward(w1, b1, w2, b2, w3, b3, wf1, bf1, wf2, bf2, x)


# speedup vs baseline: 2.0584x; 1.0322x over previous
import functools

import jax
import jax.numpy as jnp
import numpy as np
from jax.experimental import pallas as pl
from jax.experimental.pallas import tpu as pltpu

TB = 2048
R_Y1, R_A1, R_Y2, R_A2 = 28 * 168, 14 * 96, 10 * 160, 5 * 80

_C1_BASES = [28 * min(max(2 * hp - 2, 0), 22) for hp in range(14)]


def _lenet_mxu_kernel(x_ref, w1s_ref, b1_ref, p1_ref,
                      w2_ref, b2_ref, p2_ref, w3_ref, b3_ref,
                      wf1_ref, bf1_ref, wf2_ref, bf2_ref, out_ref,
                      xg_ref, y1_ref, a1_ref, y2_ref, a2_ref):
    f32 = jnp.float32
    bf16 = jnp.bfloat16
    dot = functools.partial(jnp.dot, preferred_element_type=f32)

    for y in range(28):
        xg_ref[y * 28:(y + 1) * 28, :] = x_ref[0, y]

    for hp in range(14):
        m_ref = w1s_ref[0 if hp == 0 else (2 if hp == 13 else 1)]
        xs = xg_ref[_C1_BASES[hp]:_C1_BASES[hp] + 168, :]
        y1_ref[hp * 336:(hp + 1) * 336, :] = jnp.tanh(
            dot(m_ref, xs) + b1_ref[...]).astype(bf16)

    for h2 in range(14):
        rs = (y1_ref[(2 * h2) * 168:(2 * h2 + 1) * 168, :]
              + y1_ref[(2 * h2 + 1) * 168:(2 * h2 + 2) * 168, :])
        a1_ref[h2 * 96:(h2 + 1) * 96, :] = dot(p1_ref[...], rs).astype(bf16)

    for h in range(10):
        s = a1_ref[h * 96:h * 96 + 480, :]
        y2_ref[h * 160:(h + 1) * 160, :] = jnp.tanh(
            dot(w2_ref[...], s) + b2_ref[...]).astype(bf16)

    for h2 in range(5):
        rs = (y2_ref[(2 * h2) * 160:(2 * h2 + 1) * 160, :]
              + y2_ref[(2 * h2 + 1) * 160:(2 * h2 + 2) * 160, :])
        a2_ref[h2 * 80:(h2 + 1) * 80, :] = dot(p2_ref[...], rs)

    y3 = jnp.tanh(dot(w3_ref[...], a2_ref[...]) + b3_ref[...])
    hfc = jnp.tanh(dot(wf1_ref[...], y3) + bf1_ref[...])
    out_ref[...] = dot(wf2_ref[...], hfc) + bf2_ref[...]


def _np_conv1_placement(hp):
    base_row = _C1_BASES[hp] // 28
    dh, dw, hh, w = np.meshgrid(np.arange(5), np.arange(5), np.arange(2),
                                np.arange(28), indexing="ij")
    y = 2 * hp + hh + dh - 2
    x = w + dw - 2
    ok = (y >= 0) & (y <= 27) & (x >= 0) & (x <= 27)
    c = np.zeros((5, 5, 2, 28, 6, 28), np.float32)
    rb = np.clip(y - base_row, 0, 5)
    xc = np.clip(x, 0, 27)
    np.add.at(c, (dh, dw, hh, w, rb, xc), ok.astype(np.float32))
    return c


_C1_PLACE = np.stack([_np_conv1_placement(hp) for hp in (0, 6, 13)])

_C2_PLACE = np.zeros((5, 10, 16), np.float32)
for _dw in range(5):
    for _w in range(10):
        _C2_PLACE[_dw, _w, _w + _dw] = 1.0

_P1M = np.zeros((96, 168), np.float32)
for _c in range(6):
    for _w2 in range(14):
        for _j in range(2):
            _P1M[_c * 16 + _w2, _c * 28 + 2 * _w2 + _j] = 0.25
_P2M = np.zeros((80, 160), np.float32)
for _o in range(16):
    for _w2 in range(5):
        for _j in range(2):
            _P2M[_o * 5 + _w2, _o * 10 + 2 * _w2 + _j] = 0.25


def _build_matrices(w1, b1, w2, b2, w3):
    w1r = w1.reshape(5, 5, 6)
    w1s = jnp.einsum("dec,pdehwrx->phcwrx", w1r, jnp.asarray(_C1_PLACE)
                     ).reshape(3, 336, 168).astype(jnp.bfloat16)
    b1v = jnp.tile(jnp.repeat(b1, 28), 2).reshape(336, 1)

    w2r = w2.reshape(5, 5, 6, 16)
    w2m = jnp.einsum("deco,ewx->owdcx", w2r, jnp.asarray(_C2_PLACE)
                     ).reshape(160, 480).astype(jnp.bfloat16)
    b2v = jnp.repeat(b2, 10).reshape(160, 1)

    w3p = w3.reshape(120, 16, 5, 5).transpose(0, 2, 1, 3).reshape(120, 400)
    return (w1s, b1v, jnp.asarray(_P1M, jnp.bfloat16), w2m, b2v,
            jnp.asarray(_P2M, jnp.bfloat16), w3p)


@jax.jit
def _forward(w1, b1, w2, b2, w3, b3, wf1, bf1, wf2, bf2, x):
    B = x.shape[0]
    G = (B + TB - 1) // TB
    Bp = G * TB

    (w1s, b1v, p1m, w2m, b2v, p2m, w3p
     ) = _build_matrices(w1, b1, w2, b2, w3)

    xi = jnp.transpose(x.astype(jnp.bfloat16), (1, 2, 3, 0))
    if Bp != B:
        xi = jnp.pad(xi, ((0, 0), (0, 0), (0, 0), (0, Bp - B)))

    def fixed(a):
        zeros = (0,) * a.ndim
        return pl.BlockSpec(a.shape, lambda g, _z=zeros: _z)

    consts = (w1s, b1v, p1m, w2m, b2v, p2m, w3p,
              b3, wf1, bf1, wf2, bf2)

    out = pl.pallas_call(
        _lenet_mxu_kernel,
        out_shape=jax.ShapeDtypeStruct((10, Bp), jnp.float32),
        grid=(G,),
        in_specs=[pl.BlockSpec((1, 28, 28, TB), lambda g: (0, 0, 0, g))]
        + [fixed(a) for a in consts],
        out_specs=pl.BlockSpec((10, TB), lambda g: (0, g)),
        scratch_shapes=[
            pltpu.VMEM((28 * 28, TB), jnp.bfloat16),
            pltpu.VMEM((R_Y1, TB), jnp.bfloat16),
            pltpu.VMEM((R_A1, TB), jnp.bfloat16),
            pltpu.VMEM((R_Y2, TB), jnp.bfloat16),
            pltpu.VMEM((R_A2, TB), jnp.float32),
        ],
        compiler_params=pltpu.CompilerParams(
            dimension_semantics=("parallel",),
            vmem_limit_bytes=64 * 1024 * 1024),
        cost_estimate=pl.CostEstimate(
            flops=2 * Bp * (336 * 168 * 14 + 96 * 168 * 14 + 160 * 480 * 10
                            + 80 * 160 * 5 + 120 * 400 + 84 * 120 + 10 * 84),
            transcendentals=Bp * (R_Y1 + R_Y2 + 120 + 84),
            bytes_accessed=int(Bp * 784 * 4 + Bp * 10 * 4 + 600 * 1024)),
    )(xi, *consts)

    return out.T[:B]


def kernel(w1, b1, w2, b2, w3, b3, wf1, bf1, wf2, bf2, d1, d2, x):
    del d1, d2
    return _forward(w1, b1, w2, b2, w3, b3, wf1, bf1, wf2, bf2, x)
